# baseline (speedup 1.0000x reference)
import functools

import jax
import jax.numpy as jnp
from jax import lax
from jax.experimental import pallas as pl
from jax.experimental.pallas import tpu as pltpu
from jax.experimental.pallas import tpu_sc as plsc

NUM_EMBEDDINGS = 1024
EMBEDDING_DIM = 64
COMMITMENT_COST = 0.25

_BLOCK_ROWS = 1024
_N_TOKENS = 16 * 576
_TBL_W = 2 * EMBEDDING_DIM


def _argmin_kernel(x_ref, cbp_ref, lam_ref, lamr_ref, a2_ref, idx_ref,
                   lamsel_ref):
    x = x_ref[...]
    cb = cbp_ref[:, 0:EMBEDDING_DIM]
    cbn = cbp_ref[:, EMBEDDING_DIM:2 * EMBEDDING_DIM]
    lam = lam_ref[...]

    dcb = (1.0 - lam) * cb + lam * cbn
    b2 = jnp.sum(dcb * dcb, axis=1)
    col = jax.lax.broadcasted_iota(jnp.int32, (1, NUM_EMBEDDINGS), 1)
    b2 = jnp.where(col[0] == NUM_EMBEDDINGS - 1, jnp.float32(1e30), b2)

    a2 = a2_ref[...]
    m = jax.lax.dot_general(
        x, dcb, (((1,), (1,)), ((), ())),
        preferred_element_type=jnp.float32)
    scores = jnp.sqrt(jnp.maximum((a2 + b2[None, :]) - 2.0 * m, 0.0))

    mn = jnp.min(scores, axis=1, keepdims=True)
    cols = jax.lax.broadcasted_iota(jnp.int32, scores.shape, 1)
    idx = jnp.min(jnp.where(scores == mn, cols, NUM_EMBEDDINGS),
                  axis=1, keepdims=True)
    idx_ref[...] = idx
    lamsel_ref[...] = jnp.sum(
        jnp.where(cols == idx, lamr_ref[...], 0.0), axis=1, keepdims=True)


def _make_sc_gather(n, b_per_w, nc, ns):
    mesh = plsc.VectorSubcoreMesh(core_axis_name="c", subcore_axis_name="s")

    @functools.partial(
        pl.kernel, mesh=mesh,
        out_type=jax.ShapeDtypeStruct((n, _TBL_W), jnp.float32),
        scratch_types=[
            pltpu.VMEM((b_per_w,), jnp.int32),
            pltpu.VMEM((b_per_w, _TBL_W), jnp.float32),
            pltpu.SemaphoreType.DMA,
        ],
    )
    def sc_gather(table_hbm, idx_hbm, out_hbm, idx_v, rows_v, sem):
        wid = lax.axis_index("s") * nc + lax.axis_index("c")
        base = wid * b_per_w
        pltpu.sync_copy(idx_hbm.at[pl.ds(base, b_per_w)], idx_v)
        pltpu.async_copy(table_hbm.at[idx_v], rows_v, sem).wait()
        pltpu.sync_copy(rows_v, out_hbm.at[pl.ds(base, b_per_w)])

    return sc_gather


def _combine_kernel(x_ref, g_ref, lam_ref, zq_ref, loss_ref):
    i = pl.program_id(0)
    x = x_ref[...]
    g = g_ref[...]
    c_i = g[:, 0:EMBEDDING_DIM]
    c_ip1 = g[:, EMBEDDING_DIM:2 * EMBEDDING_DIM]
    lam_i = lam_ref[...]

    d_i = c_i - x
    d_ip1 = c_ip1 - x
    n_i = jnp.sqrt(jnp.sum(d_i * d_i, axis=1, keepdims=True))
    n_ip1 = jnp.sqrt(jnp.sum(d_ip1 * d_ip1, axis=1, keepdims=True))
    s_i = n_i / (n_i + 1e-8)
    s_ip1 = n_ip1 / (n_ip1 + 1e-8)
    zq_ref[...] = x + (1.0 - lam_i) * d_i * s_i + lam_i * d_ip1 * s_ip1

    dt = (1.0 - lam_i) * c_i + lam_i * c_ip1
    r = dt - x
    part = (jnp.sum(r * r) * jnp.float32(
        (1.0 + COMMITMENT_COST) / (_N_TOKENS * EMBEDDING_DIM))).reshape(1, 1)

    @pl.when(i == 0)
    def _():
        loss_ref[...] = part

    @pl.when(i != 0)
    def _():
        loss_ref[...] += part


@jax.jit
def kernel(z, lambda_pairs, codebook):
    n = z.shape[0] * z.shape[1]
    flat = z.reshape(n, EMBEDDING_DIM)
    cb_next = jnp.concatenate([codebook[1:], codebook[:1]], axis=0)
    lam_pad = jnp.concatenate(
        [lambda_pairs, jnp.zeros((1, 1), jnp.float32)], axis=0)
    cbp = jnp.concatenate([codebook, cb_next], axis=1)
    lam_row = lam_pad.reshape(1, NUM_EMBEDDINGS)
    a2 = jnp.sum(flat ** 2, axis=1, keepdims=True)

    grid = n // _BLOCK_ROWS
    idx, lam_sel = pl.pallas_call(
        _argmin_kernel,
        grid=(grid,),
        in_specs=[
            pl.BlockSpec((_BLOCK_ROWS, EMBEDDING_DIM), lambda i: (i, 0)),
            pl.BlockSpec((NUM_EMBEDDINGS, 2 * EMBEDDING_DIM),
                         lambda i: (0, 0)),
            pl.BlockSpec((NUM_EMBEDDINGS, 1), lambda i: (0, 0)),
            pl.BlockSpec((1, NUM_EMBEDDINGS), lambda i: (0, 0)),
            pl.BlockSpec((_BLOCK_ROWS, 1), lambda i: (i, 0)),
        ],
        out_specs=[
            pl.BlockSpec((_BLOCK_ROWS, 1), lambda i: (i, 0)),
            pl.BlockSpec((_BLOCK_ROWS, 1), lambda i: (i, 0)),
        ],
        out_shape=[
            jax.ShapeDtypeStruct((n, 1), jnp.int32),
            jax.ShapeDtypeStruct((n, 1), jnp.float32),
        ],
    )(flat, cbp, lam_pad, lam_row, a2)

    info = plsc.get_sparse_core_info()
    nw = info.num_cores * info.num_subcores
    g = _make_sc_gather(n, n // nw, info.num_cores, info.num_subcores)(
        cbp, idx.reshape(n))

    zq, loss = pl.pallas_call(
        _combine_kernel,
        grid=(grid,),
        in_specs=[
            pl.BlockSpec((_BLOCK_ROWS, EMBEDDING_DIM), lambda i: (i, 0)),
            pl.BlockSpec((_BLOCK_ROWS, _TBL_W), lambda i: (i, 0)),
            pl.BlockSpec((_BLOCK_ROWS, 1), lambda i: (i, 0)),
        ],
        out_specs=[
            pl.BlockSpec((_BLOCK_ROWS, EMBEDDING_DIM), lambda i: (i, 0)),
            pl.BlockSpec((1, 1), lambda i: (0, 0)),
        ],
        out_shape=[
            jax.ShapeDtypeStruct((n, EMBEDDING_DIM), jnp.float32),
            jax.ShapeDtypeStruct((1, 1), jnp.float32),
        ],
    )(flat, g, lam_sel)

    return (zq.reshape(z.shape), loss[0, 0],
            idx[:, 0].reshape(z.shape[:-1]))
